# Initial kernel scaffold; baseline (speedup 1.0000x reference)
import jax
import jax.numpy as jnp
from jax.experimental import pallas as pl


def kernel(x, table, W, b, fi_rank):
    raise NotImplementedError("write your pallas kernel here")


# speedup vs baseline: 3.7207x; 3.7207x over previous
import jax
import jax.numpy as jnp
from jax import lax
from jax.experimental import pallas as pl
from jax.experimental.pallas import tpu as pltpu
from jax.experimental.pallas import tpu_sc as plsc

_FIELD = 26
_K = 16
_NPAIR = _FIELD * (_FIELD - 1) // 2


def _tc_body(e_ref, w_ref, b_ref, r_ref, out_ref, v_ref):
    for f in range(_FIELD):
        vf = lax.dot_general(w_ref[...], e_ref[f], (((1,), (1,)), ((), ())),
                             preferred_element_type=jnp.float32)
        v_ref[f] = vf + b_ref[...]
    off = 0
    for r in range(_FIELD - 1):
        n = _FIELD - 1 - r
        vr = v_ref[r] * r_ref[...]
        rest = v_ref[pl.ds(r + 1, n)]
        out_ref[pl.ds(off, n)] = jnp.sum(rest * vr[None, :, :], axis=1)
        off += n


def _tc_pairs(e3, W, b2, r2, bb):
    F, B, K = e3.shape
    return pl.pallas_call(
        _tc_body,
        grid=(B // bb,),
        in_specs=[
            pl.BlockSpec((F, bb, K), lambda i: (0, i, 0)),
            pl.BlockSpec((K, K), lambda i: (0, 0)),
            pl.BlockSpec((K, 1), lambda i: (0, 0)),
            pl.BlockSpec((K, 1), lambda i: (0, 0)),
        ],
        out_specs=pl.BlockSpec((_NPAIR, bb), lambda i: (0, i)),
        out_shape=jax.ShapeDtypeStruct((_NPAIR, B), jnp.float32),
        scratch_shapes=[pltpu.VMEM((F, K, bb), jnp.float32)],
    )(e3, W, b2, r2)


def _sc_gather(table, idx_flat):
    n = idx_flat.shape[0]
    info = plsc.get_sparse_core_info()
    nc, ns = info.num_cores, info.num_subcores
    nw = nc * ns
    per_w = n // nw
    chunk = 128
    nchunk = per_w // chunk
    mesh = plsc.VectorSubcoreMesh(core_axis_name="c", subcore_axis_name="s")

    def body(table_hbm, idx_hbm, out_hbm, idx_v, rows_v, sem):
        wid = lax.axis_index("s") * nc + lax.axis_index("c")
        base = wid * per_w
        pltpu.sync_copy(idx_hbm.at[pl.ds(base, per_w)], idx_v)
        handles = []
        for j in range(nchunk):
            handles.append(pltpu.async_copy(
                table_hbm.at[idx_v.at[pl.ds(j * chunk, chunk)]],
                rows_v.at[pl.ds(j * chunk, chunk), :], sem))
        for h in handles:
            h.wait()
        pltpu.sync_copy(rows_v, out_hbm.at[pl.ds(base, per_w), :])

    f = pl.kernel(
        body,
        out_type=jax.ShapeDtypeStruct((n, _K), jnp.float32),
        mesh=mesh,
        compiler_params=pltpu.CompilerParams(use_tc_tiling_on_sc=False),
        scratch_types=[
            pltpu.VMEM((per_w,), jnp.int32),
            pltpu.VMEM((per_w, _K), jnp.float32),
            pltpu.SemaphoreType.DMA,
        ],
    )
    return f(table, idx_flat)


def kernel(x, table, W, b, fi_rank):
    B, F = x.shape
    idx = x.T.reshape(-1)
    rows = _sc_gather(table, idx)
    e3 = rows.reshape(F, B, _K)
    outT = _tc_pairs(e3, W, b.reshape(_K, 1), fi_rank.reshape(_K, 1), 512)
    return outT.T
